# baseline (speedup 1.0000x reference)
import functools

import jax
import jax.numpy as jnp
from jax import lax
from jax.experimental import pallas as pl
from jax.experimental.pallas import tpu as pltpu
from jax.experimental.pallas import tpu_sc as plsc

_NC = 2
_NS = 16
_NW = _NC * _NS


def _sc_degrees(src_r, dst_r, n_pad):
  _, nblk, K = src_r.shape
  zchunk = n_pad // _NS
  NBR = 5
  nround = nblk // NBR

  mesh = plsc.VectorSubcoreMesh(core_axis_name="c", subcore_axis_name="s")

  @functools.partial(
      pl.kernel,
      out_type=[
          jax.ShapeDtypeStruct((_NC, n_pad), jnp.float32),
          jax.ShapeDtypeStruct((_NC, n_pad), jnp.float32),
      ],
      mesh=mesh,
      scratch_types=[
          pltpu.VMEM((nblk, K), jnp.int32),
          pltpu.VMEM((nblk, K), jnp.int32),
          pltpu.VMEM((K,), jnp.float32),
          pltpu.VMEM((zchunk,), jnp.float32),
          pltpu.VMEM_SHARED((n_pad,), jnp.float32),
          pltpu.VMEM_SHARED((n_pad,), jnp.float32),
          pltpu.SemaphoreType.DMA,
          pltpu.SemaphoreType.DMA,
      ],
  )
  def k(src_h, dst_h, dego_h, degi_h, idx_o, idx_i, ones_v, zbuf,
        acco_s, acci_s, semA, semB):
    c = lax.axis_index("c")
    s = lax.axis_index("s")
    wid = s * _NC + c

    @pl.loop(0, zchunk // 16)
    def _(i):
      zbuf[pl.ds(i * 16, 16)] = jnp.zeros((16,), jnp.float32)

    @pl.loop(0, K // 16)
    def _(i):
      ones_v[pl.ds(i * 16, 16)] = jnp.ones((16,), jnp.float32)

    pltpu.sync_copy(zbuf, acco_s.at[pl.ds(s * zchunk, zchunk)])
    pltpu.sync_copy(zbuf, acci_s.at[pl.ds(s * zchunk, zchunk)])
    pltpu.sync_copy(src_h.at[wid], idx_o)
    pltpu.sync_copy(dst_h.at[wid], idx_i)
    plsc.subcore_barrier()

    @pl.loop(0, nblk)
    def _(j):
      pltpu.sync_copy(ones_v, acco_s.at[idx_o.at[j]], add=True)
      pltpu.sync_copy(ones_v, acci_s.at[idx_i.at[j]], add=True)

    plsc.subcore_barrier()
    pltpu.sync_copy(acco_s.at[pl.ds(s * zchunk, zchunk)],
                    dego_h.at[c, pl.ds(s * zchunk, zchunk)])
    pltpu.sync_copy(acci_s.at[pl.ds(s * zchunk, zchunk)],
                    degi_h.at[c, pl.ds(s * zchunk, zchunk)])

  return k(src_r, dst_r)


def _sc_aggregate(h, src, dst, n_pad):
  E = src.shape[0]
  D = h.shape[1]
  per_w = (E - 3 * 80) // _NW
  K = 80
  nblk = per_w // K
  NB = 4
  nround = nblk // NB
  ntail = nblk - nround * NB
  npair = (nround - 1) // 2
  rchunk = n_pad // _NS

  mesh = plsc.VectorSubcoreMesh(core_axis_name="c", subcore_axis_name="s")

  @functools.partial(
      pl.kernel,
      out_type=jax.ShapeDtypeStruct((_NC, n_pad, D), jnp.float32),
      mesh=mesh,
      scratch_types=[
          [pltpu.VMEM((K,), jnp.int32)] * NB,
          [pltpu.VMEM((K,), jnp.int32)] * NB,
          [pltpu.VMEM((K, D), jnp.float32)] * NB,
          [pltpu.SemaphoreType.DMA] * NB,
          [pltpu.SemaphoreType.DMA] * NB,
          [pltpu.SemaphoreType.DMA] * NB,
          pltpu.VMEM_SHARED((n_pad, D), jnp.float32),
      ],
  )
  def k(h_h, src_h, dst_h, out_h, sidx, didx, rows, isem, gsem, ssem, acc_s):
    c = lax.axis_index("c")
    s = lax.axis_index("s")
    wid = s * _NC + c
    base = wid * per_w

    @pl.loop(0, K)
    def _(i):
      for j in range(D // 16):
        rows[0][i, pl.ds(j * 16, 16)] = jnp.zeros((16,), jnp.float32)

    @pl.loop(0, rchunk // K)
    def _(q):
      pltpu.sync_copy(rows[0], acc_s.at[pl.ds(s * rchunk + q * K, K)])

    plsc.subcore_barrier()

    def load_idx(j, b):
      pltpu.async_copy(src_h.at[pl.ds(base + j * K, K)], sidx[b], isem[b])
      pltpu.async_copy(dst_h.at[pl.ds(base + j * K, K)], didx[b], isem[b])

    def wait_idx(b):
      pltpu.make_async_copy(src_h.at[pl.ds(0, K)], sidx[b], isem[b]).wait()
      pltpu.make_async_copy(dst_h.at[pl.ds(0, K)], didx[b], isem[b]).wait()

    def gather(b):
      pltpu.async_copy(h_h.at[sidx[b]], rows[b], gsem[b])

    def wait_gather(b):
      pltpu.make_async_copy(h_h.at[sidx[b]], rows[b], gsem[b]).wait()

    def scatter(b):
      pltpu.async_copy(rows[b], acc_s.at[didx[b]], ssem[b], add=True)

    def wait_scatter(b):
      pltpu.make_async_copy(rows[b], acc_s.at[didx[b]], ssem[b]).wait()

    for b in range(NB):
      load_idx(b, b)
    for b in range(NB):
      wait_idx(b)
      gather(b)

    @pl.loop(0, nround - 1)
    def _(r):
      for b in range(NB):
        wait_gather(b)
        scatter(b)
      for b in range(NB):
        wait_scatter(b)
        load_idx((r + 1) * NB + b, b)
      for b in range(NB):
        wait_idx(b)
        gather(b)

    for b in range(NB):
      wait_gather(b)
      scatter(b)
    for b in range(NB):
      wait_scatter(b)

    for t in range(ntail):
      j = nround * NB + t
      load_idx(j, 0)
      wait_idx(0)
      gather(0)
      wait_gather(0)
      scatter(0)
      wait_scatter(0)

    plsc.subcore_barrier()
    pltpu.sync_copy(acc_s.at[pl.ds(s * rchunk, rchunk)],
                    out_h.at[c, pl.ds(s * rchunk, rchunk)])

  return k(h, src, dst)


def _norm_from_deg(deg_parts):
  d = deg_parts[0] + deg_parts[1]
  return jnp.where(d > 0, lax.rsqrt(jnp.maximum(d, 1.0)), 0.0)


def _tc_scale_matmul(x, W, dego, n_pad):
  B = 1024
  grid = n_pad // B
  D = x.shape[1]

  def body(x_ref, w_ref, dg_ref, o_ref):
    norm = _norm_from_deg(dg_ref[...])
    o_ref[...] = jnp.dot(x_ref[...] * norm, w_ref[...],
                         preferred_element_type=jnp.float32)

  return pl.pallas_call(
      body,
      grid=(grid,),
      in_specs=[
          pl.BlockSpec((B, D), lambda i: (i, 0)),
          pl.BlockSpec((D, D), lambda i: (0, 0)),
          pl.BlockSpec((_NC, B, 1), lambda i: (0, i, 0)),
      ],
      out_specs=pl.BlockSpec((B, D), lambda i: (i, 0)),
      out_shape=jax.ShapeDtypeStruct((n_pad, D), jnp.float32),
  )(x, W, dego)


def _tc_combine_relu_matmul(aggp, degi, dego, b, W, n_pad):
  B = 1024
  grid = n_pad // B
  D = aggp.shape[2]

  def body(p_ref, di_ref, do_ref, b_ref, w_ref, o_ref):
    a = p_ref[0] + p_ref[1]
    ni = _norm_from_deg(di_ref[...])
    o1 = jnp.maximum(a * ni + b_ref[...], 0.0)
    no = _norm_from_deg(do_ref[...])
    o_ref[...] = jnp.dot(o1 * no, w_ref[...],
                         preferred_element_type=jnp.float32)

  return pl.pallas_call(
      body,
      grid=(grid,),
      in_specs=[
          pl.BlockSpec((_NC, B, D), lambda i: (0, i, 0)),
          pl.BlockSpec((_NC, B, 1), lambda i: (0, i, 0)),
          pl.BlockSpec((_NC, B, 1), lambda i: (0, i, 0)),
          pl.BlockSpec((1, D), lambda i: (0, 0)),
          pl.BlockSpec((D, D), lambda i: (0, 0)),
      ],
      out_specs=pl.BlockSpec((B, D), lambda i: (i, 0)),
      out_shape=jax.ShapeDtypeStruct((n_pad, D), jnp.float32),
  )(aggp, degi, dego, b, W)


def _tc_combine_relu(aggp, degi, b, n_pad):
  B = 1024
  grid = n_pad // B
  D = aggp.shape[2]

  def body(p_ref, di_ref, b_ref, o_ref):
    a = p_ref[0] + p_ref[1]
    ni = _norm_from_deg(di_ref[...])
    o_ref[...] = jnp.maximum(a * ni + b_ref[...], 0.0)

  return pl.pallas_call(
      body,
      grid=(grid,),
      in_specs=[
          pl.BlockSpec((_NC, B, D), lambda i: (0, i, 0)),
          pl.BlockSpec((_NC, B, 1), lambda i: (0, i, 0)),
          pl.BlockSpec((1, D), lambda i: (0, 0)),
      ],
      out_specs=pl.BlockSpec((B, D), lambda i: (i, 0)),
      out_shape=jax.ShapeDtypeStruct((n_pad, D), jnp.float32),
  )(aggp, degi, b)


def kernel(x, edge_index, W1, b1, W2, b2):
  n, D = x.shape
  n_pad = 10240
  E = edge_index.shape[1]
  K = 80
  nblk = E // _NW // K
  src = edge_index[0]
  dst = edge_index[1]
  src_r = src.reshape(_NW, nblk, K)
  dst_r = dst.reshape(_NW, nblk, K)
  src_p = jnp.pad(src, (0, 3 * K))
  dst_p = jnp.pad(dst, (0, 3 * K))

  dego_p, degi_p = _sc_degrees(src_r, dst_r, n_pad)
  dego = dego_p.reshape(_NC, n_pad, 1)
  degi = degi_p.reshape(_NC, n_pad, 1)

  x_pad = jnp.pad(x, ((0, n_pad - n), (0, 0)))
  b1r = b1.reshape(1, D)
  b2r = b2.reshape(1, D)

  h1 = _tc_scale_matmul(x_pad, W1, dego, n_pad)
  agg1 = _sc_aggregate(h1, src_p, dst_p, n_pad)
  h2 = _tc_combine_relu_matmul(agg1, degi, dego, b1r, W2, n_pad)
  agg2 = _sc_aggregate(h2, src_p, dst_p, n_pad)
  out = _tc_combine_relu(agg2, degi, b2r, n_pad)
  return out[:n]


# speedup vs baseline: 1.0647x; 1.0647x over previous
import functools

import jax
import jax.numpy as jnp
from jax import lax
from jax.experimental import pallas as pl
from jax.experimental.pallas import tpu as pltpu
from jax.experimental.pallas import tpu_sc as plsc

_NC = 2
_NS = 16
_NW = _NC * _NS


def _sc_degrees(src_r, dst_r, n_pad):
  _, nblk, K = src_r.shape
  zchunk = n_pad // _NS
  NBR = 5
  nround = nblk // NBR

  mesh = plsc.VectorSubcoreMesh(core_axis_name="c", subcore_axis_name="s")

  @functools.partial(
      pl.kernel,
      out_type=[
          jax.ShapeDtypeStruct((_NC, n_pad), jnp.float32),
          jax.ShapeDtypeStruct((_NC, n_pad), jnp.float32),
      ],
      mesh=mesh,
      scratch_types=[
          pltpu.VMEM((nblk, K), jnp.int32),
          pltpu.VMEM((nblk, K), jnp.int32),
          pltpu.VMEM((K,), jnp.float32),
          pltpu.VMEM((zchunk,), jnp.float32),
          pltpu.VMEM_SHARED((n_pad,), jnp.float32),
          pltpu.VMEM_SHARED((n_pad,), jnp.float32),
          [pltpu.SemaphoreType.DMA] * 4,
      ],
  )
  def k(src_h, dst_h, dego_h, degi_h, idx_o, idx_i, ones_v, zbuf,
        acco_s, acci_s, isem):
    c = lax.axis_index("c")
    s = lax.axis_index("s")
    wid = s * _NC + c

    @pl.loop(0, zchunk // 16)
    def _(i):
      zbuf[pl.ds(i * 16, 16)] = jnp.zeros((16,), jnp.float32)

    @pl.loop(0, K // 16)
    def _(i):
      ones_v[pl.ds(i * 16, 16)] = jnp.ones((16,), jnp.float32)

    pltpu.sync_copy(zbuf, acco_s.at[pl.ds(s * zchunk, zchunk)])
    pltpu.sync_copy(zbuf, acci_s.at[pl.ds(s * zchunk, zchunk)])
    pltpu.sync_copy(src_h.at[wid], idx_o)
    pltpu.sync_copy(dst_h.at[wid], idx_i)
    plsc.subcore_barrier()

    NB = 4
    nround = nblk // NB
    ntail = nblk - nround * NB

    def fire(j, b):
      pltpu.async_copy(ones_v, acco_s.at[idx_o.at[j]], isem[b], add=True)
      pltpu.async_copy(ones_v, acci_s.at[idx_i.at[j]], isem[b], add=True)

    def drain(b):
      pltpu.make_async_copy(ones_v, acco_s.at[idx_o.at[0]], isem[b]).wait()
      pltpu.make_async_copy(ones_v, acci_s.at[idx_i.at[0]], isem[b]).wait()

    for b in range(NB):
      fire(b, b)

    @pl.loop(0, nround - 1)
    def _(r):
      for b in range(NB):
        drain(b)
        fire((r + 1) * NB + b, b)

    for b in range(NB):
      drain(b)
    for t in range(ntail):
      fire(nround * NB + t, 0)
      drain(0)

    plsc.subcore_barrier()
    pltpu.sync_copy(acco_s.at[pl.ds(s * zchunk, zchunk)],
                    dego_h.at[c, pl.ds(s * zchunk, zchunk)])
    pltpu.sync_copy(acci_s.at[pl.ds(s * zchunk, zchunk)],
                    degi_h.at[c, pl.ds(s * zchunk, zchunk)])

  return k(src_r, dst_r)


def _sc_aggregate(h, src, dst, n_pad):
  E = src.shape[0]
  D = h.shape[1]
  per_w = (E - 3 * 80) // _NW
  K = 80
  nblk = per_w // K
  NB = 4
  nround = nblk // NB
  ntail = nblk - nround * NB
  npair = (nround - 1) // 2
  rchunk = n_pad // _NS

  mesh = plsc.VectorSubcoreMesh(core_axis_name="c", subcore_axis_name="s")

  @functools.partial(
      pl.kernel,
      out_type=jax.ShapeDtypeStruct((_NC, n_pad, D), jnp.float32),
      mesh=mesh,
      scratch_types=[
          [pltpu.VMEM((K,), jnp.int32)] * NB,
          [pltpu.VMEM((K,), jnp.int32)] * NB,
          [pltpu.VMEM((K, D), jnp.float32)] * NB,
          [pltpu.SemaphoreType.DMA] * NB,
          [pltpu.SemaphoreType.DMA] * NB,
          [pltpu.SemaphoreType.DMA] * NB,
          pltpu.VMEM_SHARED((n_pad, D), jnp.float32),
      ],
  )
  def k(h_h, src_h, dst_h, out_h, sidx, didx, rows, isem, gsem, ssem, acc_s):
    c = lax.axis_index("c")
    s = lax.axis_index("s")
    wid = s * _NC + c
    base = wid * per_w

    @pl.loop(0, K)
    def _(i):
      for j in range(D // 16):
        rows[0][i, pl.ds(j * 16, 16)] = jnp.zeros((16,), jnp.float32)

    @pl.loop(0, rchunk // K)
    def _(q):
      pltpu.sync_copy(rows[0], acc_s.at[pl.ds(s * rchunk + q * K, K)])

    plsc.subcore_barrier()

    def load_idx(j, b):
      pltpu.async_copy(src_h.at[pl.ds(base + j * K, K)], sidx[b], isem[b])
      pltpu.async_copy(dst_h.at[pl.ds(base + j * K, K)], didx[b], isem[b])

    def wait_idx(b):
      pltpu.make_async_copy(src_h.at[pl.ds(0, K)], sidx[b], isem[b]).wait()
      pltpu.make_async_copy(dst_h.at[pl.ds(0, K)], didx[b], isem[b]).wait()

    def gather(b):
      pltpu.async_copy(h_h.at[sidx[b]], rows[b], gsem[b])

    def wait_gather(b):
      pltpu.make_async_copy(h_h.at[sidx[b]], rows[b], gsem[b]).wait()

    def scatter(b):
      pltpu.async_copy(rows[b], acc_s.at[didx[b]], ssem[b], add=True)

    def wait_scatter(b):
      pltpu.make_async_copy(rows[b], acc_s.at[didx[b]], ssem[b]).wait()

    for b in range(NB):
      load_idx(b, b)
    for b in range(NB):
      wait_idx(b)
      gather(b)

    @pl.loop(0, nround - 1)
    def _(r):
      for b in range(NB):
        wait_gather(b)
        scatter(b)
      for b in range(NB):
        wait_scatter(b)
        load_idx((r + 1) * NB + b, b)
      for b in range(NB):
        wait_idx(b)
        gather(b)

    for b in range(NB):
      wait_gather(b)
      scatter(b)
    for b in range(NB):
      wait_scatter(b)

    for t in range(ntail):
      j = nround * NB + t
      load_idx(j, 0)
      wait_idx(0)
      gather(0)
      wait_gather(0)
      scatter(0)
      wait_scatter(0)

    plsc.subcore_barrier()
    pltpu.sync_copy(acc_s.at[pl.ds(s * rchunk, rchunk)],
                    out_h.at[c, pl.ds(s * rchunk, rchunk)])

  return k(h, src, dst)


def _norm_from_deg(deg_parts):
  d = deg_parts[0] + deg_parts[1]
  return jnp.where(d > 0, lax.rsqrt(jnp.maximum(d, 1.0)), 0.0)


def _tc_scale_matmul(x, W, dego, n_pad):
  B = 1024
  grid = n_pad // B
  D = x.shape[1]

  def body(x_ref, w_ref, dg_ref, o_ref):
    norm = _norm_from_deg(dg_ref[...])
    o_ref[...] = jnp.dot(x_ref[...] * norm, w_ref[...],
                         preferred_element_type=jnp.float32)

  return pl.pallas_call(
      body,
      grid=(grid,),
      in_specs=[
          pl.BlockSpec((B, D), lambda i: (i, 0)),
          pl.BlockSpec((D, D), lambda i: (0, 0)),
          pl.BlockSpec((_NC, B, 1), lambda i: (0, i, 0)),
      ],
      out_specs=pl.BlockSpec((B, D), lambda i: (i, 0)),
      out_shape=jax.ShapeDtypeStruct((n_pad, D), jnp.float32),
  )(x, W, dego)


def _tc_combine_relu_matmul(aggp, degi, dego, b, W, n_pad):
  B = 1024
  grid = n_pad // B
  D = aggp.shape[2]

  def body(p_ref, di_ref, do_ref, b_ref, w_ref, o_ref):
    a = p_ref[0] + p_ref[1]
    ni = _norm_from_deg(di_ref[...])
    o1 = jnp.maximum(a * ni + b_ref[...], 0.0)
    no = _norm_from_deg(do_ref[...])
    o_ref[...] = jnp.dot(o1 * no, w_ref[...],
                         preferred_element_type=jnp.float32)

  return pl.pallas_call(
      body,
      grid=(grid,),
      in_specs=[
          pl.BlockSpec((_NC, B, D), lambda i: (0, i, 0)),
          pl.BlockSpec((_NC, B, 1), lambda i: (0, i, 0)),
          pl.BlockSpec((_NC, B, 1), lambda i: (0, i, 0)),
          pl.BlockSpec((1, D), lambda i: (0, 0)),
          pl.BlockSpec((D, D), lambda i: (0, 0)),
      ],
      out_specs=pl.BlockSpec((B, D), lambda i: (i, 0)),
      out_shape=jax.ShapeDtypeStruct((n_pad, D), jnp.float32),
  )(aggp, degi, dego, b, W)


def _tc_combine_relu(aggp, degi, b, n_pad):
  B = 1024
  grid = n_pad // B
  D = aggp.shape[2]

  def body(p_ref, di_ref, b_ref, o_ref):
    a = p_ref[0] + p_ref[1]
    ni = _norm_from_deg(di_ref[...])
    o_ref[...] = jnp.maximum(a * ni + b_ref[...], 0.0)

  return pl.pallas_call(
      body,
      grid=(grid,),
      in_specs=[
          pl.BlockSpec((_NC, B, D), lambda i: (0, i, 0)),
          pl.BlockSpec((_NC, B, 1), lambda i: (0, i, 0)),
          pl.BlockSpec((1, D), lambda i: (0, 0)),
      ],
      out_specs=pl.BlockSpec((B, D), lambda i: (i, 0)),
      out_shape=jax.ShapeDtypeStruct((n_pad, D), jnp.float32),
  )(aggp, degi, b)


def kernel(x, edge_index, W1, b1, W2, b2):
  n, D = x.shape
  n_pad = 10240
  E = edge_index.shape[1]
  K = 80
  nblk = E // _NW // K
  src = edge_index[0]
  dst = edge_index[1]
  src_r = src.reshape(_NW, nblk, K)
  dst_r = dst.reshape(_NW, nblk, K)
  src_p = jnp.pad(src, (0, 3 * K))
  dst_p = jnp.pad(dst, (0, 3 * K))

  dego_p, degi_p = _sc_degrees(src_r, dst_r, n_pad)
  dego = dego_p.reshape(_NC, n_pad, 1)
  degi = degi_p.reshape(_NC, n_pad, 1)

  x_pad = jnp.pad(x, ((0, n_pad - n), (0, 0)))
  b1r = b1.reshape(1, D)
  b2r = b2.reshape(1, D)

  h1 = _tc_scale_matmul(x_pad, W1, dego, n_pad)
  agg1 = _sc_aggregate(h1, src_p, dst_p, n_pad)
  h2 = _tc_combine_relu_matmul(agg1, degi, dego, b1r, W2, n_pad)
  agg2 = _sc_aggregate(h2, src_p, dst_p, n_pad)
  out = _tc_combine_relu(agg2, degi, b2r, n_pad)
  return out[:n]
